# Initial kernel scaffold; baseline (speedup 1.0000x reference)
import jax
import jax.numpy as jnp
from jax.experimental import pallas as pl


def kernel(x, edge_index, batch, W1, b1, W2, b2, W3, b3, Wl1, bl1, Wl2, bl2):
    raise NotImplementedError("write your pallas kernel here")


# speedup vs baseline: 7.1739x; 7.1739x over previous
import jax
import jax.numpy as jnp
from jax import lax
from jax.experimental import pallas as pl
from jax.experimental.pallas import tpu as pltpu
from jax.experimental.pallas import tpu_sc as plsc

_N, _E, _D, _B, _C = 10000, 320000, 128, 64, 40
_NPAD = 10240
_NC, _NS = 2, 16
_NW = _NC * _NS
_NCH = 80
_EPAD = _NW * _NCH * 128
_K0 = 48
_K1 = 2 * _NCH - _K0
_KMAX = max(_K0, _K1)
_RPS = _NPAD // _NS
_GRID = _NPAD // 512

_MESH = dict(core_axis_name="c", subcore_axis_name="s")


def _sc_deg(dsts, ones16, zeros16, iden):
    def body(d_h, o_h, z_h, i_h, out_h, idx_c, ones_v, zrows, buf, acc, sem):
        c = lax.axis_index("c")
        s = lax.axis_index("s")
        w = s * _NC + c
        pltpu.sync_copy(o_h, ones_v)
        pltpu.sync_copy(z_h.at[pl.ds(0, 128)], zrows)

        def zchunk(z, carry):
            pltpu.sync_copy(i_h.at[s, z], idx_c)
            pltpu.sync_copy(zrows, acc.at[idx_c])
            return carry

        lax.fori_loop(0, _RPS // 128, zchunk, 0)
        plsc.subcore_barrier()

        def chunk(i, carry):
            pltpu.sync_copy(d_h.at[w, i], idx_c)
            pltpu.sync_copy(ones_v, acc.at[idx_c], add=True)
            return carry

        @pl.when(c == 0)
        def _():
            lax.fori_loop(0, _K0, chunk, 0)

        @pl.when(c == 1)
        def _():
            lax.fori_loop(0, _K1, chunk, 0)
        plsc.subcore_barrier()

        def rchunk(z, carry):
            pltpu.sync_copy(i_h.at[s, z], idx_c)
            pltpu.async_copy(acc.at[idx_c], buf, sem).wait()
            pltpu.sync_copy(buf, out_h.at[c, pl.ds(s * _RPS + z * 128, 128)])
            return carry

        lax.fori_loop(0, _RPS // 128, rchunk, 0)

    f = pl.kernel(
        body,
        out_type=jax.ShapeDtypeStruct((_NC, _NPAD, 16), jnp.float32),
        mesh=plsc.VectorSubcoreMesh(**_MESH),
        scratch_types=[
            pltpu.VMEM((128,), jnp.int32),
            pltpu.VMEM((128, 16), jnp.float32),
            pltpu.VMEM((128, 16), jnp.float32),
            pltpu.VMEM((128, 16), jnp.float32),
            pltpu.MemorySpace.VMEM_SHARED((_NPAD, 16), jnp.float32),
            pltpu.SemaphoreType.DMA,
        ],
    )
    return f(dsts, ones16, zeros16, iden)


def _sc_scatter(g, srcs, dsts, zeros, iden):
    def body(g_h, s_h, d_h, z_h, i_h, out_h, idx_s, ic0, ic1, r0, r1, acc,
             si0, si1, sg0, sg1, ss0, ss1):
        c = lax.axis_index("c")
        s = lax.axis_index("s")
        w = s * _NC + c
        pltpu.sync_copy(s_h.at[w], idx_s)
        pltpu.sync_copy(z_h.at[pl.ds(0, 128)], r0)

        def zchunk(z, carry):
            pltpu.sync_copy(i_h.at[s, z], ic0)
            pltpu.sync_copy(r0, acc.at[ic0])
            return carry

        lax.fori_loop(0, _RPS // 128, zchunk, 0)
        plsc.subcore_barrier()

        def pair(j, carry):
            a = 2 * j
            b = a + 1

            @pl.when(j > 0)
            def _():
                pltpu.make_async_copy(z_h.at[pl.ds(0, 128)], r0, ss0).wait()
                pltpu.make_async_copy(z_h.at[pl.ds(0, 128)], r1, ss1).wait()

            ia = pltpu.async_copy(d_h.at[w, a], ic0, si0)
            ga = pltpu.async_copy(g_h.at[idx_s.at[a]], r0, sg0)
            ib = pltpu.async_copy(d_h.at[w, b], ic1, si1)
            gb = pltpu.async_copy(g_h.at[idx_s.at[b]], r1, sg1)
            ga.wait()
            ia.wait()
            pltpu.async_copy(r0, acc.at[ic0], ss0, add=True)
            gb.wait()
            ib.wait()
            pltpu.async_copy(r1, acc.at[ic1], ss1, add=True)
            return carry

        @pl.when(c == 0)
        def _():
            lax.fori_loop(0, _K0 // 2, pair, 0)

        @pl.when(c == 1)
        def _():
            lax.fori_loop(0, _K1 // 2, pair, 0)

        pltpu.make_async_copy(z_h.at[pl.ds(0, 128)], r0, ss0).wait()
        pltpu.make_async_copy(z_h.at[pl.ds(0, 128)], r1, ss1).wait()
        plsc.subcore_barrier()

        def rchunk(z, carry):
            pltpu.sync_copy(i_h.at[s, z], ic0)
            pltpu.async_copy(acc.at[ic0], r0, sg0).wait()
            pltpu.sync_copy(r0, out_h.at[c, pl.ds(s * _RPS + z * 128, 128)])
            return carry

        lax.fori_loop(0, _RPS // 128, rchunk, 0)

    f = pl.kernel(
        body,
        out_type=jax.ShapeDtypeStruct((_NC, _NPAD, _D), jnp.float32),
        mesh=plsc.VectorSubcoreMesh(**_MESH),
        scratch_types=[
            pltpu.VMEM((_KMAX, 128), jnp.int32),
            pltpu.VMEM((128,), jnp.int32),
            pltpu.VMEM((128,), jnp.int32),
            pltpu.VMEM((128, _D), jnp.float32),
            pltpu.VMEM((128, _D), jnp.float32),
            pltpu.MemorySpace.VMEM_SHARED((_NPAD, _D), jnp.float32),
            pltpu.SemaphoreType.DMA,
            pltpu.SemaphoreType.DMA,
            pltpu.SemaphoreType.DMA,
            pltpu.SemaphoreType.DMA,
            pltpu.SemaphoreType.DMA,
            pltpu.SemaphoreType.DMA,
        ],
    )
    return f(g, srcs, dsts, zeros, iden)


def _dinv_block(dv):
    return jnp.broadcast_to(dv[:, 0:1], (512, _D))


def _tc_first(degP, x, W1):
    def body(degp_ref, x_ref, w_ref, dinv_ref, g_ref):
        deg = degp_ref[0] + degp_ref[1] + 1.0
        dinv = lax.rsqrt(deg)
        dinv_ref[...] = dinv
        db = _dinv_block(dinv)
        g_ref[...] = jnp.dot(x_ref[...] * db, w_ref[...],
                             preferred_element_type=jnp.float32)

    return pl.pallas_call(
        body,
        grid=(_GRID,),
        in_specs=[
            pl.BlockSpec((2, 512, 16), lambda i: (0, i, 0)),
            pl.BlockSpec((512, _D), lambda i: (i, 0)),
            pl.BlockSpec((_D, _D), lambda i: (0, 0)),
        ],
        out_specs=[
            pl.BlockSpec((512, 16), lambda i: (i, 0)),
            pl.BlockSpec((512, _D), lambda i: (i, 0)),
        ],
        out_shape=[
            jax.ShapeDtypeStruct((_NPAD, 16), jnp.float32),
            jax.ShapeDtypeStruct((_NPAD, _D), jnp.float32),
        ],
    )(degP, x, W1)


def _tc_layer(accP, gprev, dinv16, bvec, W):
    def body(acc_ref, g_ref, dv_ref, b_ref, w_ref, out_ref):
        db = _dinv_block(dv_ref[...])
        h = acc_ref[0] + acc_ref[1] + g_ref[...]
        h = jnp.maximum(h * db + b_ref[...], 0.0)
        out_ref[...] = jnp.dot(h * db, w_ref[...],
                               preferred_element_type=jnp.float32)

    return pl.pallas_call(
        body,
        grid=(_GRID,),
        in_specs=[
            pl.BlockSpec((2, 512, _D), lambda i: (0, i, 0)),
            pl.BlockSpec((512, _D), lambda i: (i, 0)),
            pl.BlockSpec((512, 16), lambda i: (i, 0)),
            pl.BlockSpec((1, _D), lambda i: (0, 0)),
            pl.BlockSpec((_D, _D), lambda i: (0, 0)),
        ],
        out_specs=pl.BlockSpec((512, _D), lambda i: (i, 0)),
        out_shape=jax.ShapeDtypeStruct((_NPAD, _D), jnp.float32),
    )(accP, gprev, dinv16, bvec, W)


def _tc_pool(accP, g3, dinv16, bvec, batch16):
    def body(acc_ref, g_ref, dv_ref, b_ref, bt_ref, out_ref):
        i = pl.program_id(0)
        db = _dinv_block(dv_ref[...])
        h = acc_ref[0] + acc_ref[1] + g_ref[...]
        h = jnp.maximum(h * db + b_ref[...], 0.0)
        lanes = lax.broadcasted_iota(jnp.int32, (512, _D), 1)
        bc = jnp.broadcast_to(bt_ref[:, 0:1], (512, _D))
        oh = (bc == lanes).astype(jnp.float32)
        onescol = (lanes == 0).astype(jnp.float32)
        haug = jnp.concatenate([h, onescol], axis=1)
        S = lax.dot_general(oh, haug, (((0,), (0,)), ((), ())),
                            preferred_element_type=jnp.float32)

        @pl.when(i == 0)
        def _():
            out_ref[...] = jnp.zeros_like(out_ref)

        out_ref[...] += S

    return pl.pallas_call(
        body,
        grid=(_GRID,),
        in_specs=[
            pl.BlockSpec((2, 512, _D), lambda i: (0, i, 0)),
            pl.BlockSpec((512, _D), lambda i: (i, 0)),
            pl.BlockSpec((512, 16), lambda i: (i, 0)),
            pl.BlockSpec((1, _D), lambda i: (0, 0)),
            pl.BlockSpec((512, 16), lambda i: (i, 0)),
        ],
        out_specs=pl.BlockSpec((_D, 2 * _D), lambda i: (0, 0)),
        out_shape=jax.ShapeDtypeStruct((_D, 2 * _D), jnp.float32),
    )(accP, g3, dinv16, bvec, batch16)


def _tc_head(Saug, Wl1, bl1, Wl2p, bl2p):
    def body(s_ref, w1_ref, b1_ref, w2_ref, b2_ref, out_ref):
        sums = s_ref[:, :_D]
        cnt = jnp.broadcast_to(s_ref[:, _D:_D + 1], (_D, _D))
        pooled = sums / jnp.maximum(cnt, 1.0)
        z = jnp.maximum(
            jnp.dot(pooled, w1_ref[...], preferred_element_type=jnp.float32)
            + b1_ref[...], 0.0)
        z2 = jnp.dot(z, w2_ref[...], preferred_element_type=jnp.float32) \
            + b2_ref[...]
        m = jnp.max(z2, axis=1, keepdims=True)
        e = jnp.exp(z2 - m)
        lse = jnp.log(jnp.sum(e, axis=1, keepdims=True)) + m
        out_ref[...] = z2 - lse

    return pl.pallas_call(
        body,
        out_shape=jax.ShapeDtypeStruct((_D, _D), jnp.float32),
    )(Saug, Wl1, bl1, Wl2p, bl2p)


def kernel(x, edge_index, batch, W1, b1, W2, b2, W3, b3, Wl1, bl1, Wl2, bl2):
    xp = jnp.pad(x, ((0, _NPAD - _N), (0, 0)))
    def _split(flat, fill):
        a = flat.reshape(_NS, 2 * _NCH, 128)
        c0 = jnp.pad(a[:, :_K0], ((0, 0), (0, _KMAX - _K0), (0, 0)),
                     constant_values=fill)
        c1 = a[:, _K0:]
        return jnp.stack([c0, c1], axis=1).reshape(_NW, _KMAX, 128)

    src = _split(jnp.pad(edge_index[0], (0, _EPAD - _E)), 0)
    dst = _split(jnp.pad(edge_index[1], (0, _EPAD - _E),
                         constant_values=_NPAD - 1), _NPAD - 1)
    z128 = jnp.zeros((_NPAD, _D), jnp.float32)
    z16 = jnp.zeros((_NPAD, 16), jnp.float32)
    ones16 = jnp.ones((128, 16), jnp.float32)
    bt = jnp.broadcast_to(
        jnp.pad(batch, (0, _NPAD - _N), constant_values=_B)[:, None],
        (_NPAD, 16))

    iden = jnp.arange(_NPAD, dtype=jnp.int32).reshape(_NS, _RPS // 128, 128)
    degP = _sc_deg(dst, ones16, z16, iden)
    dinv16, g = _tc_first(degP, xp, W1)
    accP = _sc_scatter(g, src, dst, z128, iden)
    g = _tc_layer(accP, g, dinv16, b1.reshape(1, _D), W2)
    accP = _sc_scatter(g, src, dst, z128, iden)
    g = _tc_layer(accP, g, dinv16, b2.reshape(1, _D), W3)
    accP = _sc_scatter(g, src, dst, z128, iden)
    Saug = _tc_pool(accP, g, dinv16, b3.reshape(1, _D), bt)

    w2p = jnp.pad(Wl2, ((0, 0), (0, _D - _C)))
    b2p = jnp.concatenate(
        [bl2, jnp.full((_D - _C,), -1e30, jnp.float32)]).reshape(1, _D)
    logits = _tc_head(Saug, Wl1, bl1.reshape(1, _D), w2p, b2p)
    return (logits[:_B, :_C], 0)
